# baseline (speedup 1.0000x reference)
import jax
import jax.numpy as jnp
from jax.experimental import pallas as pl
from jax.experimental.pallas import tpu as pltpu

_CHUNK = 128
_Q = 8


def _gate_kernel(x_hbm, wt_ref, out_ref, x_buf, sems):
    i = pl.program_id(0)
    n = pl.num_programs(0)

    def copy(c, slot):
        return pltpu.make_async_copy(
            x_hbm.at[pl.ds(c * _CHUNK, _CHUNK), :],
            x_buf.at[slot],
            sems.at[slot],
        )

    @pl.when(i == 0)
    def _():
        for c in range(_Q):
            copy(c, c).start()

    slot = jax.lax.rem(i, _Q)
    copy(i, slot).wait()
    logits = jnp.dot(x_buf[slot], wt_ref[...],
                     preferred_element_type=jnp.float32,
                     precision=jax.lax.Precision.DEFAULT)
    m = jnp.max(logits, axis=1, keepdims=True)
    e = jnp.exp(logits - m)
    out_ref[...] = e / jnp.sum(e, axis=1, keepdims=True)

    @pl.when(i + _Q < n)
    def _():
        copy(i + _Q, slot).start()


def kernel(inputs, W):
    tokens, d = inputs.shape
    n_exp = W.shape[0]
    wt = W.T
    return pl.pallas_call(
        _gate_kernel,
        grid=(tokens // _CHUNK,),
        in_specs=[
            pl.BlockSpec(memory_space=pl.ANY),
            pl.BlockSpec((d, n_exp), lambda i: (0, 0)),
        ],
        out_specs=pl.BlockSpec((_CHUNK, n_exp), lambda i: (i, 0)),
        out_shape=jax.ShapeDtypeStruct((tokens, n_exp), jnp.float32),
        scratch_shapes=[
            pltpu.VMEM((_Q, _CHUNK, d), jnp.float32),
            pltpu.SemaphoreType.DMA((_Q,)),
        ],
        compiler_params=pltpu.CompilerParams(
            dimension_semantics=("arbitrary",),
        ),
    )(inputs, wt)


# speedup vs baseline: 1.0395x; 1.0395x over previous
import jax
import jax.numpy as jnp
from jax.experimental import pallas as pl
from jax.experimental.pallas import tpu as pltpu

_ROWS = 512
_NBUF = 3
_SUB = 4


def _gate_kernel(x_hbm, wt_ref, out_ref, x_buf, sems):
    i = pl.program_id(0)
    n = pl.num_programs(0)
    sub_rows = _ROWS // _SUB

    def copies(block, slot):
        return [
            pltpu.make_async_copy(
                x_hbm.at[pl.ds(block * _ROWS + j * sub_rows, sub_rows), :],
                x_buf.at[slot, pl.ds(j * sub_rows, sub_rows), :],
                sems.at[slot],
            )
            for j in range(_SUB)
        ]

    @pl.when(i == 0)
    def _():
        for b in range(_NBUF):
            for cp in copies(b, b):
                cp.start()

    slot = jax.lax.rem(i, _NBUF)
    for cp in copies(i, slot):
        cp.wait()
    logits = jnp.dot(x_buf[slot], wt_ref[...],
                     preferred_element_type=jnp.float32,
                     precision=jax.lax.Precision.DEFAULT)
    m = jnp.max(logits, axis=1, keepdims=True)
    e = jnp.exp(logits - m)
    out_ref[...] = e / jnp.sum(e, axis=1, keepdims=True)

    @pl.when(i + _NBUF < n)
    def _():
        for cp in copies(i + _NBUF, slot):
            cp.start()


def kernel(inputs, W):
    tokens, d = inputs.shape
    n_exp = W.shape[0]
    wt = W.T
    return pl.pallas_call(
        _gate_kernel,
        grid=(tokens // _ROWS,),
        in_specs=[
            pl.BlockSpec(memory_space=pl.ANY),
            pl.BlockSpec((d, n_exp), lambda i: (0, 0)),
        ],
        out_specs=pl.BlockSpec((_ROWS, n_exp), lambda i: (i, 0)),
        out_shape=jax.ShapeDtypeStruct((tokens, n_exp), jnp.float32),
        scratch_shapes=[
            pltpu.VMEM((_NBUF, _ROWS, d), jnp.float32),
            pltpu.SemaphoreType.DMA((_NBUF,)),
        ],
        compiler_params=pltpu.CompilerParams(
            dimension_semantics=("arbitrary",),
        ),
    )(inputs, wt)
